# Initial kernel scaffold; baseline (speedup 1.0000x reference)
import jax
import jax.numpy as jnp
from jax.experimental import pallas as pl


def kernel(input, weight):
    raise NotImplementedError("write your pallas kernel here")


# speedup vs baseline: 3.9472x; 3.9472x over previous
import functools
import math

import jax
import jax.numpy as jnp
from jax import lax
from jax.experimental import pallas as pl
from jax.experimental.pallas import tpu as pltpu
from jax.experimental.pallas import tpu_sc as plsc

DIM = 64
SCALE = (1.0 / math.sqrt(DIM)) * math.sqrt(DIM)
EPS = 1e-12

NC = 2
NS = 16
NW = NC * NS

ROW_BLOCK = 5000
CHUNK = 512


def _normalize_body(w_ref, o_ref):
    w = w_ref[...]
    ss = jnp.sum(w * w, axis=1, keepdims=True)
    norm = jnp.maximum(jnp.sqrt(ss), EPS)
    o_ref[...] = w * (SCALE / norm)


def _normalize(weight):
    v = weight.shape[0]
    return pl.pallas_call(
        _normalize_body,
        out_shape=jax.ShapeDtypeStruct(weight.shape, weight.dtype),
        grid=(v // ROW_BLOCK,),
        in_specs=[pl.BlockSpec((ROW_BLOCK, DIM), lambda i: (i, 0))],
        out_specs=pl.BlockSpec((ROW_BLOCK, DIM), lambda i: (i, 0)),
    )(weight)


def _make_gather(batch):
    b_per_w = batch // NW
    n_chunks = b_per_w // CHUNK
    mesh = plsc.VectorSubcoreMesh(
        core_axis_name="c", subcore_axis_name="s",
        num_cores=NC, num_subcores=NS)

    @functools.partial(
        pl.kernel,
        out_type=jax.ShapeDtypeStruct((batch, DIM), jnp.float32),
        mesh=mesh,
        scratch_types=[
            pltpu.VMEM((b_per_w,), jnp.int32),
            pltpu.VMEM((2, CHUNK, DIM), jnp.float32),
            pltpu.SemaphoreType.DMA,
            pltpu.SemaphoreType.DMA,
        ],
        compiler_params=pltpu.CompilerParams(use_tc_tiling_on_sc=False),
    )
    def gather(table_hbm, idx_hbm, out_hbm, idx_v, rows_v, gsem, ssem):
        wid = lax.axis_index("s") * NC + lax.axis_index("c")
        base = wid * b_per_w
        pltpu.sync_copy(idx_hbm.at[pl.ds(base, b_per_w)], idx_v)

        pltpu.async_copy(
            table_hbm.at[idx_v.at[pl.ds(0, CHUNK)]], rows_v.at[0], gsem)

        @pl.loop(0, n_chunks, step=2)
        def _(g):
            for b in range(2):
                cur = g + b
                pltpu.make_async_copy(
                    table_hbm.at[idx_v.at[pl.ds(0, CHUNK)]],
                    rows_v.at[b], gsem).wait()
                @pl.when(cur + 1 < n_chunks)
                def _():
                    pltpu.async_copy(
                        table_hbm.at[idx_v.at[pl.ds((cur + 1) * CHUNK, CHUNK)]],
                        rows_v.at[1 - b], gsem)
                pltpu.async_copy(
                    rows_v.at[b],
                    out_hbm.at[pl.ds(base + cur * CHUNK, CHUNK)], ssem).wait()

    return gather


def kernel(input, weight):
    w_normed = _normalize(weight)
    idx = input.reshape(-1).astype(jnp.int32)
    out = _make_gather(idx.shape[0])(w_normed, idx)
    return out.reshape(input.shape + (DIM,))
